# baseline (speedup 1.0000x reference)
import jax
import jax.numpy as jnp
from jax.experimental import pallas as pl
from jax.experimental.pallas import tpu as pltpu
from jax.experimental.pallas import tpu_sc as plsc

VOCAB = 1024
DIM = 64
N_ROWS = 32 * 576
HALF = N_ROWS // 2
BLOCK = 2304
NBH = HALF // BLOCK
EBLOCK = 4608
NEH = HALF // EBLOCK
GATHER_WINDOW = 256


def _tc_body(x_ref, e_ref, a2_ref, b2_ref, idx_ref, counts_ref):
    x = x_ref[...]
    xe = jax.lax.dot_general(x, e_ref[...], (((1,), (1,)), ((), ())),
                             preferred_element_type=jnp.float32)
    d = (a2_ref[...] + b2_ref[...]) - 4.0 * xe

    mv = jnp.min(d, axis=1, keepdims=True)
    lane = jax.lax.broadcasted_iota(jnp.int32, d.shape, 1)
    idx = jnp.min(jnp.where(d == mv, lane, jnp.int32(2 ** 30)),
                  axis=1, keepdims=True)
    idx_ref[...] = idx

    onehot = lane == idx
    counts = jnp.sum(onehot.astype(jnp.float32), axis=0, keepdims=True)
    counts_ref[...] = counts[None]


def _tc_quantize(x_half, emb, a2_half, b2):
    return pl.pallas_call(
        _tc_body,
        grid=(NBH,),
        in_specs=[
            pl.BlockSpec((BLOCK, DIM), lambda i: (i, 0)),
            pl.BlockSpec((VOCAB, DIM), lambda i: (0, 0)),
            pl.BlockSpec((BLOCK, 1), lambda i: (i, 0)),
            pl.BlockSpec((1, VOCAB), lambda i: (0, 0)),
        ],
        out_specs=[
            pl.BlockSpec((BLOCK, 1), lambda i: (i, 0)),
            pl.BlockSpec((1, 1, VOCAB), lambda i: (i, 0, 0)),
        ],
        out_shape=[
            jax.ShapeDtypeStruct((HALF, 1), jnp.int32),
            jax.ShapeDtypeStruct((NBH, 1, VOCAB), jnp.float32),
        ],
    )(x_half, emb, a2_half, b2)


def _sc_gather(emb_padded, idx_flat):
    mesh = plsc.VectorSubcoreMesh(core_axis_name="core",
                                  subcore_axis_name="subcore")

    @pl.kernel(out_type=jax.ShapeDtypeStruct((HALF, 128), jnp.float32),
               mesh=mesh)
    def k(emb_hbm, i_hbm, o_hbm):
        def body(i_vmem, o_vmem):
            pltpu.sync_copy(emb_hbm.at[i_vmem.at[0]], o_vmem)

        pltpu.emit_pipeline(
            body,
            grid=(HALF // GATHER_WINDOW,),
            in_specs=[pl.BlockSpec((1, GATHER_WINDOW),
                                   index_map=lambda i: (0, i))],
            out_specs=[pl.BlockSpec((GATHER_WINDOW, 128),
                                    index_map=lambda i: (i, 0))],
            core_axis_name=("core", "subcore"),
            dimension_semantics=(pltpu.PARALLEL,),
        )(i_hbm, o_hbm)

    return k(emb_padded, idx_flat)


def _tc_epilogue_body(x_ref, qp_ref, qst_ref, se_ref):
    q = qp_ref[...][:, :DIM]
    qst_ref[...] = q
    diff = q - x_ref[...]
    se_ref[...] = jnp.zeros((1, 1, 128), jnp.float32) + jnp.sum(diff * diff)


def _tc_epilogue(x_half, qp_half):
    return pl.pallas_call(
        _tc_epilogue_body,
        grid=(NEH,),
        in_specs=[
            pl.BlockSpec((EBLOCK, DIM), lambda i: (i, 0)),
            pl.BlockSpec((EBLOCK, 128), lambda i: (i, 0)),
        ],
        out_specs=[
            pl.BlockSpec((EBLOCK, DIM), lambda i: (i, 0)),
            pl.BlockSpec((1, 1, 128), lambda i: (i, 0, 0)),
        ],
        out_shape=[
            jax.ShapeDtypeStruct((HALF, DIM), jnp.float32),
            jax.ShapeDtypeStruct((NEH, 1, 128), jnp.float32),
        ],
    )(x_half, qp_half)


def _tc_finalize_body(counts_ref, se_ref, loss_ref, perp_ref):
    counts = jnp.sum(counts_ref[...], axis=0)
    avg = counts / jnp.float32(N_ROWS)
    ent = jnp.sum(avg * jnp.log(avg + 1e-10), axis=1, keepdims=True)
    perp_ref[...] = jnp.exp(-ent)
    se = jnp.sum(se_ref[...][:, :, :1])
    loss_ref[...] = jnp.reshape(
        1.25 * se / jnp.float32(N_ROWS * DIM), (1, 1))


def _tc_finalize(counts_p, se_p):
    return pl.pallas_call(
        _tc_finalize_body,
        out_shape=[
            jax.ShapeDtypeStruct((1, 1), jnp.float32),
            jax.ShapeDtypeStruct((1, 1), jnp.float32),
        ],
    )(counts_p, se_p)


def kernel(inputs, embedding):
    input_shape = inputs.shape
    x = inputs.reshape(-1, DIM)
    a2 = jnp.sum(x ** 2, axis=1, keepdims=True)
    b2 = jnp.sum(embedding ** 2, axis=1).reshape(1, VOCAB)
    emb_padded = jnp.pad(embedding, ((0, 0), (0, 128 - DIM)))

    x0, x1 = x[:HALF], x[HALF:]
    a20, a21 = a2[:HALF], a2[HALF:]

    idx0, counts0 = _tc_quantize(x0, embedding, a20, b2)
    idx1, counts1 = _tc_quantize(x1, embedding, a21, b2)
    qp0 = _sc_gather(emb_padded, idx0.reshape(1, HALF))
    qp1 = _sc_gather(emb_padded, idx1.reshape(1, HALF))
    qst0, se0 = _tc_epilogue(x0, qp0)
    qst1, se1 = _tc_epilogue(x1, qp1)

    counts_p = jnp.concatenate([counts0, counts1], axis=0)
    se_p = jnp.concatenate([se0, se1], axis=0)
    loss, perp = _tc_finalize(counts_p, se_p)

    idx = jnp.concatenate([idx0, idx1], axis=0).reshape(-1)
    quantized_st = jnp.concatenate([qst0, qst1], axis=0).reshape(input_shape)
    enc_idx_out = idx.reshape(input_shape[0], input_shape[1])
    return (quantized_st, enc_idx_out, loss.reshape(()), perp.reshape(()))


# speedup vs baseline: 1.3753x; 1.3753x over previous
import jax
import jax.numpy as jnp
from jax.experimental import pallas as pl
from jax.experimental.pallas import tpu as pltpu
from jax.experimental.pallas import tpu_sc as plsc

VOCAB = 1024
DIM = 64
N_ROWS = 32 * 576
BLOCK = 2304
NB = N_ROWS // BLOCK
GATHER_WINDOW = 256


def _tc_body(x_ref, e_ref, a2_ref, b2_ref, idx_ref, counts_ref, sa_ref):
    x = x_ref[...]
    xe = jax.lax.dot_general(x, e_ref[...], (((1,), (1,)), ((), ())),
                             preferred_element_type=jnp.float32)
    a2 = a2_ref[...]
    d = (a2 + b2_ref[...]) - 4.0 * xe

    mv = jnp.min(d, axis=1, keepdims=True)
    lane = jax.lax.broadcasted_iota(jnp.int32, d.shape, 1)
    idx = jnp.min(jnp.where(d == mv, lane, jnp.int32(2 ** 30)),
                  axis=1, keepdims=True)
    idx_ref[...] = idx

    onehot = lane == idx
    counts = jnp.sum(onehot.astype(jnp.float32), axis=0, keepdims=True)
    counts_ref[...] = counts[None]
    sa_ref[...] = jnp.zeros((1, 1, 128), jnp.float32) + jnp.sum(a2 + mv)


def _tc_quantize(x, emb, a2, b2):
    return pl.pallas_call(
        _tc_body,
        grid=(NB,),
        in_specs=[
            pl.BlockSpec((BLOCK, DIM), lambda i: (i, 0)),
            pl.BlockSpec((VOCAB, DIM), lambda i: (0, 0)),
            pl.BlockSpec((BLOCK, 1), lambda i: (i, 0)),
            pl.BlockSpec((1, VOCAB), lambda i: (0, 0)),
        ],
        out_specs=[
            pl.BlockSpec((BLOCK, 1), lambda i: (i, 0)),
            pl.BlockSpec((1, 1, VOCAB), lambda i: (i, 0, 0)),
            pl.BlockSpec((1, 1, 128), lambda i: (i, 0, 0)),
        ],
        out_shape=[
            jax.ShapeDtypeStruct((N_ROWS, 1), jnp.int32),
            jax.ShapeDtypeStruct((NB, 1, VOCAB), jnp.float32),
            jax.ShapeDtypeStruct((NB, 1, 128), jnp.float32),
        ],
    )(x, emb, a2, b2)


def _sc_gather(emb_padded, idx_flat):
    mesh = plsc.VectorSubcoreMesh(core_axis_name="core",
                                  subcore_axis_name="subcore")

    @pl.kernel(out_type=jax.ShapeDtypeStruct((N_ROWS, 128), jnp.float32),
               mesh=mesh)
    def k(emb_hbm, i_hbm, o_hbm):
        def body(i_vmem, o_vmem):
            pltpu.sync_copy(emb_hbm.at[i_vmem.at[0]], o_vmem)

        pltpu.emit_pipeline(
            body,
            grid=(N_ROWS // GATHER_WINDOW,),
            in_specs=[pl.BlockSpec((1, GATHER_WINDOW),
                                   index_map=lambda i: (0, i))],
            out_specs=[pl.BlockSpec((GATHER_WINDOW, 128),
                                    index_map=lambda i: (i, 0))],
            core_axis_name=("core", "subcore"),
            dimension_semantics=(pltpu.PARALLEL,),
        )(i_hbm, o_hbm)

    return k(emb_padded, idx_flat)


def _tc_finalize_body(counts_ref, sa_ref, b2_ref, loss_ref, perp_ref):
    counts = jnp.sum(counts_ref[...], axis=0)
    avg = counts / jnp.float32(N_ROWS)
    ent = jnp.sum(avg * jnp.log(avg + 1e-10), axis=1, keepdims=True)
    perp_ref[...] = jnp.exp(-ent)
    b2tot = jnp.sum(counts * b2_ref[...])
    sa = jnp.sum(sa_ref[...][:, :, :1])
    se = 0.5 * (sa + b2tot)
    loss_ref[...] = jnp.reshape(
        1.25 * se / jnp.float32(N_ROWS * DIM), (1, 1))


def _tc_finalize(counts_p, sa_p, b2):
    return pl.pallas_call(
        _tc_finalize_body,
        out_shape=[
            jax.ShapeDtypeStruct((1, 1), jnp.float32),
            jax.ShapeDtypeStruct((1, 1), jnp.float32),
        ],
    )(counts_p, sa_p, b2)


def kernel(inputs, embedding):
    input_shape = inputs.shape
    x = inputs.reshape(-1, DIM)
    a2 = jnp.sum(x ** 2, axis=1, keepdims=True)
    b2 = jnp.sum(embedding ** 2, axis=1).reshape(1, VOCAB)
    emb_padded = jnp.pad(embedding, ((0, 0), (0, 128 - DIM)))

    idx2, counts_p, sa_p = _tc_quantize(x, embedding, a2, b2)
    idx = idx2.reshape(-1)
    qp = _sc_gather(emb_padded, idx.reshape(1, N_ROWS))
    loss, perp = _tc_finalize(counts_p, sa_p, b2)

    quantized_st = qp[:, :DIM].reshape(input_shape)
    enc_idx_out = idx.reshape(input_shape[0], input_shape[1])
    return (quantized_st, enc_idx_out, loss.reshape(()), perp.reshape(()))
